# baseline (speedup 1.0000x reference)
import functools

import jax
import jax.numpy as jnp
from jax import lax
from jax.experimental import pallas as pl
from jax.experimental.pallas import tpu as pltpu
from jax.experimental.pallas import tpu_sc as plsc

_B, _C, _H, _W = 64, 384, 24, 24
_CO = _C // 2
_NW = 32
_BPW = _B // _NW


def _make_sc_copy():
    mesh = plsc.VectorSubcoreMesh(core_axis_name="c", subcore_axis_name="s")

    @functools.partial(
        pl.kernel,
        mesh=mesh,
        out_type=jax.ShapeDtypeStruct((_B, _CO, _H, _W), jnp.float32),
        scratch_types=[
            pltpu.VMEM((1, 1, _H, _W), jnp.float32),
        ],
    )
    def sc_copy(in_hbm, out_hbm, buf):
        wid = lax.axis_index("s") * 2 + lax.axis_index("c")

        def body(i, carry):
            b = _BPW * wid + i // _CO
            j = i % _CO
            pltpu.sync_copy(in_hbm.at[pl.ds(b, 1), pl.ds(2 * j, 1)], buf)
            pltpu.sync_copy(buf, out_hbm.at[pl.ds(b, 1), pl.ds(j, 1)])
            return carry

        lax.fori_loop(0, _BPW * _CO, body, 0)

    return sc_copy


_sc_copy = _make_sc_copy()


def kernel(input_tensor):
    return _sc_copy(input_tensor)


# speedup vs baseline: 1.7280x; 1.7280x over previous
import functools

import jax
import jax.numpy as jnp
from jax import lax
from jax.experimental import pallas as pl
from jax.experimental.pallas import tpu as pltpu
from jax.experimental.pallas import tpu_sc as plsc

_B, _C, _H, _W = 64, 384, 24, 24
_CO = _C // 2
_NW = 32
_BPW = _B // _NW
_K = 16
_CPB = _CO // _K
_NCH = _BPW * _CPB


def _make_sc_copy():
    mesh = plsc.VectorSubcoreMesh(core_axis_name="c", subcore_axis_name="s")

    @functools.partial(
        pl.kernel,
        mesh=mesh,
        out_type=jax.ShapeDtypeStruct((_B, _CO, _H, _W), jnp.float32),
        scratch_types=[
            pltpu.VMEM((1, _K, _H, _W), jnp.float32),
            pltpu.VMEM((1, _K, _H, _W), jnp.float32),
            pltpu.SemaphoreType.DMA,
            pltpu.SemaphoreType.DMA,
            pltpu.SemaphoreType.DMA,
            pltpu.SemaphoreType.DMA,
        ],
    )
    def sc_copy(in_hbm, out_hbm, buf0, buf1, isem0, isem1, osem0, osem1):
        wid = lax.axis_index("s") * 2 + lax.axis_index("c")
        bufs = (buf0, buf1)
        isems = (isem0, isem1)
        osems = (osem0, osem1)

        def chunk_coords(c):
            b = _BPW * wid + c // _CPB
            j0 = (c % _CPB) * _K
            return b, j0

        def fire_in(c, p):
            b, j0 = chunk_coords(c)
            handles = []
            for g in range(_K):
                handles.append(pltpu.async_copy(
                    in_hbm.at[pl.ds(b, 1), pl.ds(2 * (j0 + g), 1)],
                    bufs[p].at[pl.ds(0, 1), pl.ds(g, 1)],
                    isems[p]))
            return handles

        def drain_in(c, p):
            b, j0 = chunk_coords(c)
            pltpu.make_async_copy(
                in_hbm.at[pl.ds(b, 1), pl.ds(j0, _K)], bufs[p], isems[p]
            ).wait()

        def fire_out(c, p):
            b, j0 = chunk_coords(c)
            pltpu.async_copy(
                bufs[p], out_hbm.at[pl.ds(b, 1), pl.ds(j0, _K)], osems[p])

        def drain_out(c, p):
            b, j0 = chunk_coords(c)
            pltpu.make_async_copy(
                bufs[p], out_hbm.at[pl.ds(b, 1), pl.ds(j0, _K)], osems[p]
            ).wait()

        def run_chunk(c, p, first):
            if not first:
                drain_out(c, p)
            fire_in(c, p)
            drain_in(c, p)
            fire_out(c, p)

        run_chunk(0, 0, True)
        run_chunk(1, 1, True)

        def body(cc, carry):
            c = 2 + 2 * cc
            run_chunk(c, 0, False)
            run_chunk(c + 1, 1, False)
            return carry

        lax.fori_loop(0, (_NCH - 2) // 2, body, 0)

        drain_out(_NCH - 2, 0)
        drain_out(_NCH - 1, 1)

    return sc_copy


_sc_copy = _make_sc_copy()


def kernel(input_tensor):
    return _sc_copy(input_tensor)


# speedup vs baseline: 6.1008x; 3.5305x over previous
import functools

import jax
import jax.numpy as jnp
from jax import lax
from jax.experimental import pallas as pl
from jax.experimental.pallas import tpu as pltpu
from jax.experimental.pallas import tpu_sc as plsc

_B, _C, _H, _W = 64, 384, 24, 24
_CO = _C // 2
_RTOT = _B * _H * _W
_NW = 32
_RPW = _RTOT // _NW
_R = 64
_NCH = _RPW // _R
_NQ = _CO // 16


def _make_sc_select():
    mesh = plsc.VectorSubcoreMesh(core_axis_name="c", subcore_axis_name="s")

    @functools.partial(
        pl.kernel,
        mesh=mesh,
        out_type=jax.ShapeDtypeStruct((_RTOT, _CO), jnp.float32),
        scratch_types=[
            pltpu.VMEM((_R, _C), jnp.float32),
            pltpu.VMEM((_R, _C), jnp.float32),
            pltpu.VMEM((_R, _CO), jnp.float32),
            pltpu.VMEM((_R, _CO), jnp.float32),
            pltpu.SemaphoreType.DMA,
            pltpu.SemaphoreType.DMA,
            pltpu.SemaphoreType.DMA,
            pltpu.SemaphoreType.DMA,
        ],
    )
    def sc_select(in_hbm, out_hbm, ibuf0, ibuf1, obuf0, obuf1,
                  isem0, isem1, osem0, osem1):
        wid = lax.axis_index("s") * 2 + lax.axis_index("c")
        base = wid * _RPW
        ibufs = (ibuf0, ibuf1)
        obufs = (obuf0, obuf1)
        isems = (isem0, isem1)
        osems = (osem0, osem1)

        lane = lax.iota(jnp.int32, 16)
        evens = (2 * lane) % 16
        low = lane < 8
        dnums = lax.GatherDimensionNumbers(
            offset_dims=(), collapsed_slice_dims=(0,), start_index_map=(0,))

        def compact(a, b):
            ga = lax.gather(a, evens[:, None], dnums, slice_sizes=(1,),
                            mode=lax.GatherScatterMode.PROMISE_IN_BOUNDS)
            gb = lax.gather(b, evens[:, None], dnums, slice_sizes=(1,),
                            mode=lax.GatherScatterMode.PROMISE_IN_BOUNDS)
            return jnp.where(low, ga, gb)

        def fire_in(c, p):
            pltpu.async_copy(
                in_hbm.at[pl.ds(base + c * _R, _R)], ibufs[p], isems[p])

        def drain_in(c, p):
            pltpu.make_async_copy(
                in_hbm.at[pl.ds(base + c * _R, _R)], ibufs[p], isems[p]
            ).wait()

        def fire_out(c, p):
            pltpu.async_copy(
                obufs[p], out_hbm.at[pl.ds(base + c * _R, _R)], osems[p])

        def drain_out(c, p):
            pltpu.make_async_copy(
                obufs[p], out_hbm.at[pl.ds(base + c * _R, _R)], osems[p]
            ).wait()

        def compute(p):
            ib, ob = ibufs[p], obufs[p]

            def row(r, carry):
                for q in range(_NQ):
                    a = ib[r, pl.ds(32 * q, 16)]
                    b = ib[r, pl.ds(32 * q + 16, 16)]
                    ob[r, pl.ds(16 * q, 16)] = compact(a, b)
                return carry

            lax.fori_loop(0, _R, row, 0)

        fire_in(0, 0)
        fire_in(1, 1)
        for c in range(_NCH):
            p = c % 2
            drain_in(c, p)
            if c >= 2:
                drain_out(c - 2, p)
            compute(p)
            fire_out(c, p)
            if c + 2 < _NCH:
                fire_in(c + 2, p)
        drain_out(_NCH - 2, 0)
        drain_out(_NCH - 1, 1)

    return sc_select


_sc_select = _make_sc_select()


def kernel(input_tensor):
    x = input_tensor.transpose(0, 2, 3, 1).reshape(_RTOT, _C)
    out2 = _sc_select(x)
    return out2.reshape(_B, _H, _W, _CO).transpose(0, 3, 1, 2)


# speedup vs baseline: 10.5192x; 1.7242x over previous
import functools

import jax
import jax.numpy as jnp
from jax import lax
from jax.experimental import pallas as pl
from jax.experimental.pallas import tpu as pltpu
from jax.experimental.pallas import tpu_sc as plsc

_B, _C, _H, _W = 64, 384, 24, 24
_CO = _C // 2
_RTOT = _B * _H * _W
_NW = 32
_RPW = _RTOT // _NW
_R = 64
_NCH = _RPW // _R
_NQ = _CO // 16


def _make_sc_select():
    mesh = plsc.VectorSubcoreMesh(core_axis_name="c", subcore_axis_name="s")

    @functools.partial(
        pl.kernel,
        mesh=mesh,
        out_type=jax.ShapeDtypeStruct((_RTOT, _CO), jnp.float32),
        scratch_types=[
            pltpu.VMEM((_R, _C), jnp.float32),
            pltpu.VMEM((_R, _C), jnp.float32),
            pltpu.VMEM((_R, _CO), jnp.float32),
            pltpu.VMEM((_R, _CO), jnp.float32),
            pltpu.SemaphoreType.DMA,
            pltpu.SemaphoreType.DMA,
            pltpu.SemaphoreType.DMA,
            pltpu.SemaphoreType.DMA,
        ],
    )
    def sc_select(in_hbm, out_hbm, ibuf0, ibuf1, obuf0, obuf1,
                  isem0, isem1, osem0, osem1):
        wid = lax.axis_index("s") * 2 + lax.axis_index("c")
        base = wid * _RPW
        ibufs = (ibuf0, ibuf1)
        obufs = (obuf0, obuf1)
        isems = (isem0, isem1)
        osems = (osem0, osem1)

        lane = lax.iota(jnp.int32, 16)
        evens = (2 * lane) % 16
        low = lane < 8
        dnums = lax.GatherDimensionNumbers(
            offset_dims=(), collapsed_slice_dims=(0,), start_index_map=(0,))

        def compact(a, b):
            ga = lax.gather(a, evens[:, None], dnums, slice_sizes=(1,),
                            mode=lax.GatherScatterMode.PROMISE_IN_BOUNDS)
            gb = lax.gather(b, evens[:, None], dnums, slice_sizes=(1,),
                            mode=lax.GatherScatterMode.PROMISE_IN_BOUNDS)
            return jnp.where(low, ga, gb)

        def fire_in(c, p):
            pltpu.async_copy(
                in_hbm.at[pl.ds(base + c * _R, _R)], ibufs[p], isems[p])

        def drain_in(c, p):
            pltpu.make_async_copy(
                in_hbm.at[pl.ds(base + c * _R, _R)], ibufs[p], isems[p]
            ).wait()

        def fire_out(c, p):
            pltpu.async_copy(
                obufs[p], out_hbm.at[pl.ds(base + c * _R, _R)], osems[p])

        def drain_out(c, p):
            pltpu.make_async_copy(
                obufs[p], out_hbm.at[pl.ds(base + c * _R, _R)], osems[p]
            ).wait()

        def compute(p):
            ib, ob = ibufs[p], obufs[p]

            @plsc.parallel_loop(0, _R, unroll=8)
            def _row(r):
                for q in range(_NQ):
                    a = ib[r, pl.ds(32 * q, 16)]
                    b = ib[r, pl.ds(32 * q + 16, 16)]
                    ob[r, pl.ds(16 * q, 16)] = compact(a, b)

        fire_in(0, 0)
        fire_in(1, 1)
        for c in range(_NCH):
            p = c % 2
            drain_in(c, p)
            if c >= 2:
                drain_out(c - 2, p)
            compute(p)
            fire_out(c, p)
            if c + 2 < _NCH:
                fire_in(c + 2, p)
        drain_out(_NCH - 2, 0)
        drain_out(_NCH - 1, 1)

    return sc_select


_sc_select = _make_sc_select()


def kernel(input_tensor):
    x = input_tensor.transpose(0, 2, 3, 1).reshape(_RTOT, _C)
    out2 = _sc_select(x)
    return out2.reshape(_B, _H, _W, _CO).transpose(0, 3, 1, 2)


# speedup vs baseline: 11.0985x; 1.0551x over previous
import functools

import jax
import jax.numpy as jnp
from jax import lax
from jax.experimental import pallas as pl
from jax.experimental.pallas import tpu as pltpu
from jax.experimental.pallas import tpu_sc as plsc

_B, _C, _H, _W = 64, 384, 24, 24
_CO = _C // 2
_RTOT = _B * _H * _W
_NW = 32
_RPW = _RTOT // _NW
_R = 96
_NCH = _RPW // _R
_NQ = _CO // 16


def _make_sc_select():
    mesh = plsc.VectorSubcoreMesh(core_axis_name="c", subcore_axis_name="s")

    @functools.partial(
        pl.kernel,
        mesh=mesh,
        out_type=jax.ShapeDtypeStruct((_RTOT, _CO), jnp.float32),
        scratch_types=[
            pltpu.VMEM((_R, _C), jnp.float32),
            pltpu.VMEM((_R, _C), jnp.float32),
            pltpu.VMEM((_R, _CO), jnp.float32),
            pltpu.VMEM((_R, _CO), jnp.float32),
            pltpu.SemaphoreType.DMA,
            pltpu.SemaphoreType.DMA,
            pltpu.SemaphoreType.DMA,
            pltpu.SemaphoreType.DMA,
        ],
    )
    def sc_select(in_hbm, out_hbm, ibuf0, ibuf1, obuf0, obuf1,
                  isem0, isem1, osem0, osem1):
        wid = lax.axis_index("s") * 2 + lax.axis_index("c")
        base = wid * _RPW
        ibufs = (ibuf0, ibuf1)
        obufs = (obuf0, obuf1)
        isems = (isem0, isem1)
        osems = (osem0, osem1)

        lane = lax.iota(jnp.int32, 16)
        evens = (2 * lane) % 16
        low = lane < 8
        dnums = lax.GatherDimensionNumbers(
            offset_dims=(), collapsed_slice_dims=(0,), start_index_map=(0,))

        def compact(a, b):
            ga = lax.gather(a, evens[:, None], dnums, slice_sizes=(1,),
                            mode=lax.GatherScatterMode.PROMISE_IN_BOUNDS)
            gb = lax.gather(b, evens[:, None], dnums, slice_sizes=(1,),
                            mode=lax.GatherScatterMode.PROMISE_IN_BOUNDS)
            return jnp.where(low, ga, gb)

        def fire_in(c, p):
            pltpu.async_copy(
                in_hbm.at[pl.ds(base + c * _R, _R)], ibufs[p], isems[p])

        def drain_in(c, p):
            pltpu.make_async_copy(
                in_hbm.at[pl.ds(base + c * _R, _R)], ibufs[p], isems[p]
            ).wait()

        def fire_out(c, p):
            pltpu.async_copy(
                obufs[p], out_hbm.at[pl.ds(base + c * _R, _R)], osems[p])

        def drain_out(c, p):
            pltpu.make_async_copy(
                obufs[p], out_hbm.at[pl.ds(base + c * _R, _R)], osems[p]
            ).wait()

        def compute(p):
            ib, ob = ibufs[p], obufs[p]

            @plsc.parallel_loop(0, _R, unroll=8)
            def _row(r):
                for q in range(_NQ):
                    a = ib[r, pl.ds(32 * q, 16)]
                    b = ib[r, pl.ds(32 * q + 16, 16)]
                    ob[r, pl.ds(16 * q, 16)] = compact(a, b)

        fire_in(0, 0)
        fire_in(1, 1)
        for c in range(_NCH):
            p = c % 2
            drain_in(c, p)
            if c >= 2:
                drain_out(c - 2, p)
            compute(p)
            fire_out(c, p)
            if c + 2 < _NCH:
                fire_in(c + 2, p)
        drain_out(_NCH - 2, 0)
        drain_out(_NCH - 1, 1)

    return sc_select


_sc_select = _make_sc_select()


def kernel(input_tensor):
    x = input_tensor.transpose(0, 2, 3, 1).reshape(_RTOT, _C)
    out2 = _sc_select(x)
    return out2.reshape(_B, _H, _W, _CO).transpose(0, 3, 1, 2)
